# baseline (speedup 1.0000x reference)
import jax
import jax.numpy as jnp
from jax.experimental import pallas as pl
from jax.experimental.pallas import tpu as pltpu

_B, _N, _DIM = 2, 2048, 64
_H, _DH = 8, 64
_CBS = 64
_SBS = 64
_NSEL = 8
_WIN = 128
_NB = _N // _CBS
_QT = 256
_KT = 512
_WT = _WIN + _QT
_BH = _B * _H
_DA = _DH + _NB
_PEN = 2.0 ** 30

_NT = (((1,), (1,)), ((), ()))


def _attn_body(q_ref, ka_ref, v_ref, ckf_ref, cvf_ref, g_ref, o_ref):
    i = pl.program_id(1)
    q = q_ref[0]
    scale = _DH ** -0.5
    rows = i * _QT + jax.lax.broadcasted_iota(jnp.int32, (_QT, 1), 0)

    ckf = ckf_ref[0]
    csim = jax.lax.dot_general(q, ckf, _NT,
                               preferred_element_type=jnp.float32) * scale
    col = jax.lax.broadcasted_iota(jnp.int32, (_QT, _NB + 1), 1)
    csim = jnp.where(col * _CBS - 1 <= rows, csim, -1e9)
    cmax = jnp.max(csim, axis=-1, keepdims=True)
    cexp = jnp.exp(csim - cmax)
    cattn = cexp / jnp.sum(cexp, axis=-1, keepdims=True)
    c_out = jnp.dot(cattn, cvf_ref[0], preferred_element_type=jnp.float32)

    ck1 = ckf[1:, :]
    rsim = jax.lax.dot_general(ck1, q, _NT,
                               preferred_element_type=jnp.float32) * scale
    jrow = jax.lax.broadcasted_iota(jnp.int32, (_NB, _QT), 0)
    ncol = i * _QT + jax.lax.broadcasted_iota(jnp.int32, (_NB, _QT), 1)
    rsim = jnp.where((jrow + 1) * _CBS - 1 <= ncol, rsim, -1e9)
    rsim = jnp.where(jrow == ncol // _SBS, 1e4, rsim)
    a = rsim[:, None, :]
    b = rsim[None, :, :]
    jj = jax.lax.broadcasted_iota(jnp.int32, (_NB, _NB, 1), 0)
    kk = jax.lax.broadcasted_iota(jnp.int32, (_NB, _NB, 1), 1)
    beats = (b > a) | ((b == a) & (kk < jj))
    rank = jnp.sum(beats.astype(jnp.float32), axis=1)
    selT = (rank < _NSEL - 0.5).astype(jnp.float32)
    q_aug = jnp.concatenate([q, 1.0 - selT.T], axis=1)

    nfull = i // 2

    def chunk(c, carry):
        m, l, acc = carry
        ka_c = ka_ref[0, pl.ds(c * _KT, _KT), :]
        v_c = v_ref[0, pl.ds(c * _KT, _KT), :]
        s = jax.lax.dot_general(q_aug, ka_c, _NT,
                                preferred_element_type=jnp.float32) * scale
        m_new = jnp.maximum(m, jnp.max(s, axis=-1, keepdims=True))
        p = jnp.exp(s - m_new)
        alpha = jnp.exp(m - m_new)
        l = l * alpha + jnp.sum(p, axis=-1, keepdims=True)
        acc = acc * alpha + jnp.dot(p, v_c, preferred_element_type=jnp.float32)
        return m_new, l, acc

    m0 = jnp.full((_QT, 1), -1e30, jnp.float32)
    l0 = jnp.zeros((_QT, 1), jnp.float32)
    a0 = jnp.zeros((_QT, _DH), jnp.float32)
    m, l, acc = jax.lax.fori_loop(0, nfull, chunk, (m0, l0, a0))

    dstart = nfull * _KT
    ka_d = ka_ref[0, pl.ds(dstart, _KT), :]
    v_d = v_ref[0, pl.ds(dstart, _KT), :]
    s = jax.lax.dot_general(q_aug, ka_d, _NT,
                            preferred_element_type=jnp.float32) * scale
    kpos = dstart + jax.lax.broadcasted_iota(jnp.int32, (_QT, _KT), 1)
    s = jnp.where(kpos <= rows, s, -1e9)
    m_new = jnp.maximum(m, jnp.max(s, axis=-1, keepdims=True))
    p = jnp.exp(s - m_new)
    alpha = jnp.exp(m - m_new)
    l = l * alpha + jnp.sum(p, axis=-1, keepdims=True)
    acc = acc * alpha + jnp.dot(p, v_d, preferred_element_type=jnp.float32)
    f_out = acc / l

    wstart = jnp.maximum(i * _QT - _WIN, 0)
    k_w = ka_ref[0, pl.ds(wstart, _WT), :_DH]
    v_w = v_ref[0, pl.ds(wstart, _WT), :]
    ws = jax.lax.dot_general(q, k_w, _NT,
                             preferred_element_type=jnp.float32) * scale
    wpos = wstart + jax.lax.broadcasted_iota(jnp.int32, (_QT, _WT), 1)
    wmask = (wpos <= rows) & (wpos > rows - _WIN)
    ws = jnp.where(wmask, ws, -1e9)
    wmax = jnp.max(ws, axis=-1, keepdims=True)
    wexp = jnp.exp(ws - wmax)
    s_out = jnp.dot(wexp / jnp.sum(wexp, axis=-1, keepdims=True), v_w,
                    preferred_element_type=jnp.float32)

    g = g_ref[0]
    o_ref[0] = (g[:, 0:1] * c_out + g[:, 1:2] * f_out + g[:, 2:3] * s_out)


def _fused_attention(q, ka, v, ckf, cvf, gates):
    return pl.pallas_call(
        _attn_body,
        grid=(_BH, _N // _QT),
        in_specs=[
            pl.BlockSpec((1, _QT, _DH), lambda bh, i: (bh, i, 0)),
            pl.BlockSpec((1, _N, _DA), lambda bh, i: (bh, 0, 0)),
            pl.BlockSpec((1, _N, _DH), lambda bh, i: (bh, 0, 0)),
            pl.BlockSpec((1, _NB + 1, _DH), lambda bh, i: (bh, 0, 0)),
            pl.BlockSpec((1, _NB + 1, _DH), lambda bh, i: (bh, 0, 0)),
            pl.BlockSpec((1, _QT, 3), lambda bh, i: (bh, i, 0)),
        ],
        out_specs=pl.BlockSpec((1, _QT, _DH), lambda bh, i: (bh, i, 0)),
        out_shape=jax.ShapeDtypeStruct((_BH, _N, _DH), jnp.float32),
        compiler_params=pltpu.CompilerParams(
            dimension_semantics=("parallel", "arbitrary"),
        ),
    )(q, ka, v, ckf, cvf, gates)


def kernel(x, W_emb, b_emb, g_norm, W_q, W_k, W_v, k_pos, v_pos, Wc_k, Wc_v,
           mem_ck, mem_cv, W_gate, b_gate, W_o, W1, b1, W2, b2):
    B, N, DIM, H, DH, NB, CBS = _B, _N, _DIM, _H, _DH, _NB, _CBS

    xe = x[..., None] * W_emb[0] + b_emb
    h = xe * jax.lax.rsqrt(jnp.mean(xe * xe, axis=-1, keepdims=True) + 1e-6)
    h = h * g_norm

    def proj(W):
        return (h @ W).reshape(B, N, H, DH).transpose(0, 2, 1, 3)

    q, k, v = proj(W_q), proj(W_k), proj(W_v)

    kb = k.reshape(B, H, NB, CBS, DH) + k_pos
    vb = v.reshape(B, H, NB, CBS, DH) + v_pos
    ck = kb.reshape(B, H, NB, CBS * DH) @ Wc_k
    cv = vb.reshape(B, H, NB, CBS * DH) @ Wc_v
    ckf = jnp.concatenate([jnp.broadcast_to(mem_ck, (B, H, 1, DH)), ck], axis=2)
    cvf = jnp.concatenate([jnp.broadcast_to(mem_cv, (B, H, 1, DH)), cv], axis=2)

    gates = jax.nn.sigmoid(xe @ W_gate + b_gate)
    gates = gates.reshape(B, N, 3, H).transpose(0, 3, 1, 2)

    blkflag = -_PEN * (jax.nn.one_hot(jnp.arange(N) // CBS, NB,
                                      dtype=jnp.float32))
    ka = jnp.concatenate(
        [k.reshape(_BH, N, DH),
         jnp.broadcast_to(blkflag, (_BH, N, NB))], axis=-1)

    out = _fused_attention(
        q.reshape(_BH, N, DH), ka, v.reshape(_BH, N, DH),
        ckf.reshape(_BH, NB + 1, DH), cvf.reshape(_BH, NB + 1, DH),
        gates.reshape(_BH, N, 3),
    )

    out = out.reshape(B, H, N, DH).transpose(0, 2, 1, 3).reshape(B, N, H * DH)
    out = out @ W_o
    pooled = out.mean(axis=1)
    h1 = jax.nn.gelu(pooled @ W1 + b1, approximate=False)
    return h1 @ W2 + b2


# speedup vs baseline: 1.2300x; 1.2300x over previous
import jax
import jax.numpy as jnp
from jax.experimental import pallas as pl
from jax.experimental.pallas import tpu as pltpu

_B, _N, _DIM = 2, 2048, 64
_H, _DH = 8, 64
_CBS = 64
_SBS = 64
_NSEL = 8
_WIN = 128
_NB = _N // _CBS
_QT = 256
_KT = 512
_WT = _WIN + _QT
_BH = _B * _H
_DA = _DH + _NB
_PEN = 2.0 ** 30

_NT = (((1,), (1,)), ((), ()))


def _attn_body(x_ref, wemb_ref, bemb_ref, gn_ref, wq_ref, wk_ref, wv_ref,
               kpos_ref, vpos_ref, wck_ref, wcv_ref, mck_ref, mcv_ref,
               wg_ref, bg_ref, bf_ref, o_ref,
               qs, kas, vs, ckfs, cvfs):
    i = pl.program_id(1)
    scale = _DH ** -0.5

    @pl.when(i == 0)
    def _prologue():
        x_col = x_ref[0]
        xe = x_col * wemb_ref[...] + bemb_ref[...]
        ms = jnp.dot(xe * xe, jnp.ones((_DIM, 1), jnp.float32),
                     preferred_element_type=jnp.float32) * (1.0 / _DIM)
        h = xe * jax.lax.rsqrt(ms + 1e-6) * gn_ref[...]
        qs[...] = jnp.dot(h, wq_ref[0], preferred_element_type=jnp.float32)
        k = jnp.dot(h, wk_ref[0], preferred_element_type=jnp.float32)
        v = jnp.dot(h, wv_ref[0], preferred_element_type=jnp.float32)
        vs[...] = v
        kas[...] = jnp.concatenate([k, bf_ref[0]], axis=1)
        kb = k.reshape(_NB, _CBS, _DH) + kpos_ref[...]
        vb = v.reshape(_NB, _CBS, _DH) + vpos_ref[...]
        ck = jnp.dot(kb.reshape(_NB, _CBS * _DH), wck_ref[...],
                     preferred_element_type=jnp.float32)
        cv = jnp.dot(vb.reshape(_NB, _CBS * _DH), wcv_ref[...],
                     preferred_element_type=jnp.float32)
        ckfs[...] = jnp.concatenate([mck_ref[0], ck], axis=0)
        cvfs[...] = jnp.concatenate([mcv_ref[0], cv], axis=0)

    q = qs[pl.ds(i * _QT, _QT), :]
    rows = i * _QT + jax.lax.broadcasted_iota(jnp.int32, (_QT, 1), 0)

    x_t = x_ref[0, pl.ds(i * _QT, _QT), :]
    xe_t = x_t * wemb_ref[...] + bemb_ref[...]
    g = jax.nn.sigmoid(jnp.dot(xe_t, wg_ref[0],
                               preferred_element_type=jnp.float32)
                       + bg_ref[0])

    ckf = ckfs[...]
    csim = jax.lax.dot_general(q, ckf, _NT,
                               preferred_element_type=jnp.float32) * scale
    col = jax.lax.broadcasted_iota(jnp.int32, (_QT, _NB + 1), 1)
    csim = jnp.where(col * _CBS - 1 <= rows, csim, -1e9)
    cmax = jnp.max(csim, axis=-1, keepdims=True)
    cexp = jnp.exp(csim - cmax)
    cattn = cexp / jnp.sum(cexp, axis=-1, keepdims=True)
    c_out = jnp.dot(cattn, cvfs[...], preferred_element_type=jnp.float32)

    ck1 = ckf[1:, :]
    rsim = jax.lax.dot_general(ck1, q, _NT,
                               preferred_element_type=jnp.float32) * scale
    jrow = jax.lax.broadcasted_iota(jnp.int32, (_NB, _QT), 0)
    ncol = i * _QT + jax.lax.broadcasted_iota(jnp.int32, (_NB, _QT), 1)
    rsim = jnp.where((jrow + 1) * _CBS - 1 <= ncol, rsim, -1e9)
    rsim = jnp.where(jrow == ncol // _SBS, 1e4, rsim)
    a = rsim[:, None, :]
    b = rsim[None, :, :]
    jj = jax.lax.broadcasted_iota(jnp.int32, (_NB, _NB, 1), 0)
    kk = jax.lax.broadcasted_iota(jnp.int32, (_NB, _NB, 1), 1)
    beats = (b > a) | ((b == a) & (kk < jj))
    rank = jnp.sum(beats.astype(jnp.float32), axis=1)
    selT = (rank < _NSEL - 0.5).astype(jnp.float32)
    q_aug = jnp.concatenate([q, 1.0 - selT.T], axis=1)

    nfull = i // 2

    def chunk(c, carry):
        m, l, acc = carry
        ka_c = kas[pl.ds(c * _KT, _KT), :]
        v_c = vs[pl.ds(c * _KT, _KT), :]
        s = jax.lax.dot_general(q_aug, ka_c, _NT,
                                preferred_element_type=jnp.float32) * scale
        m_new = jnp.maximum(m, jnp.max(s, axis=-1, keepdims=True))
        p = jnp.exp(s - m_new)
        alpha = jnp.exp(m - m_new)
        l = l * alpha + jnp.sum(p, axis=-1, keepdims=True)
        acc = acc * alpha + jnp.dot(p, v_c, preferred_element_type=jnp.float32)
        return m_new, l, acc

    m0 = jnp.full((_QT, 1), -1e30, jnp.float32)
    l0 = jnp.zeros((_QT, 1), jnp.float32)
    a0 = jnp.zeros((_QT, _DH), jnp.float32)
    m, l, acc = jax.lax.fori_loop(0, nfull, chunk, (m0, l0, a0))

    dstart = nfull * _KT
    ka_d = kas[pl.ds(dstart, _KT), :]
    v_d = vs[pl.ds(dstart, _KT), :]
    s = jax.lax.dot_general(q_aug, ka_d, _NT,
                            preferred_element_type=jnp.float32) * scale
    kpos = dstart + jax.lax.broadcasted_iota(jnp.int32, (_QT, _KT), 1)
    s = jnp.where(kpos <= rows, s, -1e9)
    m_new = jnp.maximum(m, jnp.max(s, axis=-1, keepdims=True))
    p = jnp.exp(s - m_new)
    alpha = jnp.exp(m - m_new)
    l = l * alpha + jnp.sum(p, axis=-1, keepdims=True)
    acc = acc * alpha + jnp.dot(p, v_d, preferred_element_type=jnp.float32)
    f_out = acc / l

    wstart = jnp.maximum(i * _QT - _WIN, 0)
    k_w = kas[pl.ds(wstart, _WT), :_DH]
    v_w = vs[pl.ds(wstart, _WT), :]
    ws = jax.lax.dot_general(q, k_w, _NT,
                             preferred_element_type=jnp.float32) * scale
    wpos = wstart + jax.lax.broadcasted_iota(jnp.int32, (_QT, _WT), 1)
    wmask = (wpos <= rows) & (wpos > rows - _WIN)
    ws = jnp.where(wmask, ws, -1e9)
    wmax = jnp.max(ws, axis=-1, keepdims=True)
    wexp = jnp.exp(ws - wmax)
    s_out = jnp.dot(wexp / jnp.sum(wexp, axis=-1, keepdims=True), v_w,
                    preferred_element_type=jnp.float32)

    o_ref[0] = (g[:, 0:1] * c_out + g[:, 1:2] * f_out + g[:, 2:3] * s_out)


def _fused_attention(x3, wemb, bemb, gn, wq3, wk3, wv3, kpos, vpos,
                     wck3, wcv3, mck, mcv, wg3, bg3, blkflag):
    batch_spec = pl.BlockSpec((1, _N, 1), lambda bh, i: (bh // _H, 0, 0))
    head_spec = pl.BlockSpec((1, _DIM, _DH), lambda bh, i: (bh % _H, 0, 0))
    full2 = lambda a: pl.BlockSpec(a.shape, lambda bh, i: (0,) * a.ndim)
    headv = lambda shp: pl.BlockSpec((1,) + shp,
                                     lambda bh, i: (bh % _H, 0, 0))
    return pl.pallas_call(
        _attn_body,
        grid=(_BH, _N // _QT),
        in_specs=[
            batch_spec,
            full2(wemb), full2(bemb), full2(gn),
            head_spec, head_spec, head_spec,
            full2(kpos), full2(vpos),
            full2(wck3), full2(wcv3),
            headv((1, _DH)), headv((1, _DH)),
            headv((_DIM, 3)), headv((1, 3)),
            full2(blkflag),
        ],
        out_specs=pl.BlockSpec((1, _QT, _DH), lambda bh, i: (bh, i, 0)),
        out_shape=jax.ShapeDtypeStruct((_BH, _N, _DH), jnp.float32),
        scratch_shapes=[
            pltpu.VMEM((_N, _DH), jnp.float32),
            pltpu.VMEM((_N, _DA), jnp.float32),
            pltpu.VMEM((_N, _DH), jnp.float32),
            pltpu.VMEM((_NB + 1, _DH), jnp.float32),
            pltpu.VMEM((_NB + 1, _DH), jnp.float32),
        ],
        compiler_params=pltpu.CompilerParams(
            dimension_semantics=("arbitrary", "arbitrary"),
        ),
    )(x3, wemb, bemb, gn, wq3, wk3, wv3, kpos, vpos,
      wck3, wcv3, mck, mcv, wg3, bg3, blkflag)


def kernel(x, W_emb, b_emb, g_norm, W_q, W_k, W_v, k_pos, v_pos, Wc_k, Wc_v,
           mem_ck, mem_cv, W_gate, b_gate, W_o, W1, b1, W2, b2):
    B, N, DIM, H, DH, NB, CBS = _B, _N, _DIM, _H, _DH, _NB, _CBS

    x3 = x.reshape(B, N, 1)
    wq3 = W_q.reshape(DIM, H, DH).transpose(1, 0, 2)
    wk3 = W_k.reshape(DIM, H, DH).transpose(1, 0, 2)
    wv3 = W_v.reshape(DIM, H, DH).transpose(1, 0, 2)
    wck3 = Wc_k
    wcv3 = Wc_v
    wg3 = W_gate.reshape(DIM, 3, H).transpose(2, 0, 1)
    bg3 = b_gate.reshape(3, H).T.reshape(H, 1, 3)
    blkflag = (-_PEN * jax.nn.one_hot(jnp.arange(N) // CBS, NB,
                                      dtype=jnp.float32)).reshape(1, N, NB)

    out = _fused_attention(
        x3, W_emb, b_emb.reshape(1, DIM), g_norm.reshape(1, DIM),
        wq3, wk3, wv3, k_pos, v_pos, wck3, wcv3, mem_ck, mem_cv,
        wg3, bg3, blkflag)

    out = out.reshape(B, H, N, DH).transpose(0, 2, 1, 3).reshape(B, N, H * DH)
    out = out @ W_o
    pooled = out.mean(axis=1)
    h1 = jax.nn.gelu(pooled @ W1 + b1, approximate=False)
    return h1 @ W2 + b2


# speedup vs baseline: 1.2406x; 1.0086x over previous
import jax
import jax.numpy as jnp
from jax.experimental import pallas as pl
from jax.experimental.pallas import tpu as pltpu

_B, _N, _DIM = 2, 2048, 64
_H, _DH = 8, 64
_CBS = 64
_SBS = 64
_NSEL = 8
_WIN = 128
_NB = _N // _CBS
_QT = 256
_KT = 512
_WT = _WIN + _QT
_BH = _B * _H
_DA = _DH + _NB
_PEN = 2.0 ** 30

_NT = (((1,), (1,)), ((), ()))


def _attn_body(x_ref, wemb_ref, bemb_ref, gn_ref, wq_ref, wk_ref, wv_ref,
               kpos_ref, vpos_ref, wck_ref, wcv_ref, mck_ref, mcv_ref,
               wg_ref, bg_ref, bf_ref, o_ref,
               qs, kas, vs, ckfs, cvfs):
    i = pl.program_id(1)
    scale = _DH ** -0.5

    @pl.when(i == 0)
    def _prologue():
        x_col = x_ref[0]
        xe = x_col * wemb_ref[...] + bemb_ref[...]
        ms = jnp.dot(xe * xe, jnp.ones((_DIM, 1), jnp.float32),
                     preferred_element_type=jnp.float32) * (1.0 / _DIM)
        h = xe * jax.lax.rsqrt(ms + 1e-6) * gn_ref[...]
        qs[...] = jnp.dot(h, wq_ref[0], preferred_element_type=jnp.float32)
        k = jnp.dot(h, wk_ref[0], preferred_element_type=jnp.float32)
        v = jnp.dot(h, wv_ref[0], preferred_element_type=jnp.float32)
        vs[...] = v
        kas[...] = jnp.concatenate([k, bf_ref[0]], axis=1)
        kb = k.reshape(_NB, _CBS, _DH) + kpos_ref[...]
        vb = v.reshape(_NB, _CBS, _DH) + vpos_ref[...]
        ck = jnp.dot(kb.reshape(_NB, _CBS * _DH), wck_ref[...],
                     preferred_element_type=jnp.float32)
        cv = jnp.dot(vb.reshape(_NB, _CBS * _DH), wcv_ref[...],
                     preferred_element_type=jnp.float32)
        ckfs[...] = jnp.concatenate([mck_ref[0], ck], axis=0)
        cvfs[...] = jnp.concatenate([mcv_ref[0], cv], axis=0)

    q = qs[pl.ds(i * _QT, _QT), :]
    rows = i * _QT + jax.lax.broadcasted_iota(jnp.int32, (_QT, 1), 0)

    x_t = x_ref[0, pl.ds(i * _QT, _QT), :]
    xe_t = x_t * wemb_ref[...] + bemb_ref[...]
    g = jax.nn.sigmoid(jnp.dot(xe_t, wg_ref[0],
                               preferred_element_type=jnp.float32)
                       + bg_ref[0])

    ckf = ckfs[...]
    csim = jax.lax.dot_general(q, ckf, _NT,
                               preferred_element_type=jnp.float32) * scale
    col = jax.lax.broadcasted_iota(jnp.int32, (_QT, _NB + 1), 1)
    csim = jnp.where(col * _CBS - 1 <= rows, csim, -1e9)
    cmax = jnp.max(csim, axis=-1, keepdims=True)
    cexp = jnp.exp(csim - cmax)
    cattn = cexp / jnp.sum(cexp, axis=-1, keepdims=True)
    c_out = jnp.dot(cattn, cvfs[...], preferred_element_type=jnp.float32)

    ck1 = ckf[1:, :]
    rsim = jax.lax.dot_general(ck1, q, _NT,
                               preferred_element_type=jnp.float32) * scale
    jrow = jax.lax.broadcasted_iota(jnp.int32, (_NB, _QT), 0)
    ncol = i * _QT + jax.lax.broadcasted_iota(jnp.int32, (_NB, _QT), 1)
    rsim = jnp.where((jrow + 1) * _CBS - 1 <= ncol, rsim, -1e9)
    rsim = jnp.where(jrow == ncol // _SBS, 1e4, rsim)
    a = rsim[:, None, :]
    b = rsim[None, :, :]
    jj = jax.lax.broadcasted_iota(jnp.int32, (_NB, _NB, 1), 0)
    kk = jax.lax.broadcasted_iota(jnp.int32, (_NB, _NB, 1), 1)
    beats = (b > a) | ((b == a) & (kk < jj))
    rank = jnp.sum(beats.astype(jnp.float32), axis=1)
    selT = (rank < _NSEL - 0.5).astype(jnp.float32)
    q_aug = jnp.concatenate([q, 1.0 - selT.T], axis=1)

    nfull = i // 2

    def chunk(c, carry):
        m, l, acc = carry
        ka_c = kas[pl.ds(c * _KT, _KT), :]
        v_c = vs[pl.ds(c * _KT, _KT), :]
        s = jax.lax.dot_general(q_aug, ka_c, _NT,
                                preferred_element_type=jnp.float32) * scale
        m_new = jnp.maximum(m, jnp.max(s, axis=-1, keepdims=True))
        p = jnp.exp(s - m_new)
        alpha = jnp.exp(m - m_new)
        l = l * alpha + jnp.sum(p, axis=-1, keepdims=True)
        acc = acc * alpha + jnp.dot(p, v_c, preferred_element_type=jnp.float32)
        return m_new, l, acc

    m0 = jnp.full((_QT, 1), -1e30, jnp.float32)
    l0 = jnp.zeros((_QT, 1), jnp.float32)
    a0 = jnp.zeros((_QT, _DH), jnp.float32)
    m, l, acc = jax.lax.fori_loop(0, nfull, chunk, (m0, l0, a0))

    dstart = nfull * _KT
    ka_d = kas[pl.ds(dstart, _KT), :]
    v_d = vs[pl.ds(dstart, _KT), :]
    s = jax.lax.dot_general(q_aug, ka_d, _NT,
                            preferred_element_type=jnp.float32) * scale
    kpos = dstart + jax.lax.broadcasted_iota(jnp.int32, (_QT, _KT), 1)
    s = jnp.where(kpos <= rows, s, -1e9)
    m_new = jnp.maximum(m, jnp.max(s, axis=-1, keepdims=True))
    p = jnp.exp(s - m_new)
    alpha = jnp.exp(m - m_new)
    l = l * alpha + jnp.sum(p, axis=-1, keepdims=True)
    acc = acc * alpha + jnp.dot(p, v_d, preferred_element_type=jnp.float32)
    f_out = acc / l

    wstart = jnp.maximum(i * _QT - _WIN, 0)
    k_w = kas[pl.ds(wstart, _WT), :_DH]
    v_w = vs[pl.ds(wstart, _WT), :]
    ws = jax.lax.dot_general(q, k_w, _NT,
                             preferred_element_type=jnp.float32) * scale
    wpos = wstart + jax.lax.broadcasted_iota(jnp.int32, (_QT, _WT), 1)
    wmask = (wpos <= rows) & (wpos > rows - _WIN)
    ws = jnp.where(wmask, ws, -1e9)
    wmax = jnp.max(ws, axis=-1, keepdims=True)
    wexp = jnp.exp(ws - wmax)
    s_out = jnp.dot(wexp / jnp.sum(wexp, axis=-1, keepdims=True), v_w,
                    preferred_element_type=jnp.float32)

    combined = g[:, 0:1] * c_out + g[:, 1:2] * f_out + g[:, 2:3] * s_out

    @pl.when(i == 0)
    def _init_out():
        o_ref[...] = jnp.zeros_like(o_ref)

    o_ref[0] = o_ref[0] + jnp.dot(jnp.ones((1, _QT), jnp.float32), combined,
                                  preferred_element_type=jnp.float32)


def _fused_attention(x3, wemb, bemb, gn, wq3, wk3, wv3, kpos, vpos,
                     wck3, wcv3, mck, mcv, wg3, bg3, blkflag):
    batch_spec = pl.BlockSpec((1, _N, 1), lambda bh, i: (bh // _H, 0, 0))
    head_spec = pl.BlockSpec((1, _DIM, _DH), lambda bh, i: (bh % _H, 0, 0))
    full2 = lambda a: pl.BlockSpec(a.shape, lambda bh, i: (0,) * a.ndim)
    headv = lambda shp: pl.BlockSpec((1,) + shp,
                                     lambda bh, i: (bh % _H, 0, 0))
    return pl.pallas_call(
        _attn_body,
        grid=(_BH, _N // _QT),
        in_specs=[
            batch_spec,
            full2(wemb), full2(bemb), full2(gn),
            head_spec, head_spec, head_spec,
            full2(kpos), full2(vpos),
            full2(wck3), full2(wcv3),
            headv((1, _DH)), headv((1, _DH)),
            headv((_DIM, 3)), headv((1, 3)),
            full2(blkflag),
        ],
        out_specs=pl.BlockSpec((1, 1, _DH), lambda bh, i: (bh, 0, 0)),
        out_shape=jax.ShapeDtypeStruct((_BH, 1, _DH), jnp.float32),
        scratch_shapes=[
            pltpu.VMEM((_N, _DH), jnp.float32),
            pltpu.VMEM((_N, _DA), jnp.float32),
            pltpu.VMEM((_N, _DH), jnp.float32),
            pltpu.VMEM((_NB + 1, _DH), jnp.float32),
            pltpu.VMEM((_NB + 1, _DH), jnp.float32),
        ],
        compiler_params=pltpu.CompilerParams(
            dimension_semantics=("arbitrary", "arbitrary"),
        ),
    )(x3, wemb, bemb, gn, wq3, wk3, wv3, kpos, vpos,
      wck3, wcv3, mck, mcv, wg3, bg3, blkflag)


def kernel(x, W_emb, b_emb, g_norm, W_q, W_k, W_v, k_pos, v_pos, Wc_k, Wc_v,
           mem_ck, mem_cv, W_gate, b_gate, W_o, W1, b1, W2, b2):
    B, N, DIM, H, DH, NB, CBS = _B, _N, _DIM, _H, _DH, _NB, _CBS

    x3 = x.reshape(B, N, 1)
    wq3 = W_q.reshape(DIM, H, DH).transpose(1, 0, 2)
    wk3 = W_k.reshape(DIM, H, DH).transpose(1, 0, 2)
    wv3 = W_v.reshape(DIM, H, DH).transpose(1, 0, 2)
    wck3 = Wc_k
    wcv3 = Wc_v
    wg3 = W_gate.reshape(DIM, 3, H).transpose(2, 0, 1)
    bg3 = b_gate.reshape(3, H).T.reshape(H, 1, 3)
    blkflag = (-_PEN * jax.nn.one_hot(jnp.arange(N) // CBS, NB,
                                      dtype=jnp.float32)).reshape(1, N, NB)

    out = _fused_attention(
        x3, W_emb, b_emb.reshape(1, DIM), g_norm.reshape(1, DIM),
        wq3, wk3, wv3, k_pos, v_pos, wck3, wcv3, mem_ck, mem_cv,
        wg3, bg3, blkflag)

    pooled = (out.reshape(B, H * DH) / N) @ W_o
    h1 = jax.nn.gelu(pooled @ W1 + b1, approximate=False)
    return h1 @ W2 + b2
